# baseline (speedup 1.0000x reference)
import functools

import jax
import jax.numpy as jnp
from jax import lax
from jax.experimental import pallas as pl
from jax.experimental.pallas import tpu as pltpu
from jax.experimental.pallas import tpu_sc as plsc

_N = 10000
_E = 320000
_F = 128
_D = 64
_C = 40
_NW = 32
_CHW = 256
_CH = 40
_CAP = _NW * _CH * _CHW
_NT = 10112
_SINK = 10100
_RPT = _NT // 16
_R = 2
_HPT = _NT // 16

_MESH = dict(
    mesh=plsc.VectorSubcoreMesh(core_axis_name="c", subcore_axis_name="s"),
    compiler_params=pltpu.CompilerParams(use_tc_tiling_on_sc=False),
)


def _edge_pass_body(tbl_hbm, pair_hbm, zeros_hbm, out_hbm,
                    idx_p, idxb, rows, acc, tbl_s, gsem, ssem, zsem):
    c = lax.axis_index("c")
    s = lax.axis_index("s")
    w = s * 2 + c
    pltpu.sync_copy(pair_hbm.at[w], idx_p)

    @pl.loop(0, _RPT // 8)
    def _(i):
        pltpu.async_copy(zeros_hbm, acc.at[pl.ds(s * _RPT + 8 * i, 8)], zsem)

    pltpu.sync_copy(tbl_hbm.at[pl.ds(s * 624, 624)],
                    tbl_s.at[pl.ds(s * 624, 624)])

    @pl.when(s == 15)
    def _():
        pltpu.sync_copy(tbl_hbm.at[pl.ds(9984, 16)],
                        tbl_s.at[pl.ds(9984, 16)])

    @pl.loop(0, _RPT // 8)
    def _(i):
        pltpu.make_async_copy(zeros_hbm, acc.at[pl.ds(s * _RPT + 8 * i, 8)],
                              zsem).wait()

    plsc.subcore_barrier()

    def unpack(r, j):
        for k in range(_CHW // 16):
            v = idx_p[j, pl.ds(16 * k, 16)]
            idxb[r, 0, pl.ds(16 * k, 16)] = jnp.bitwise_and(v, 16383)
            idxb[r, 1, pl.ds(16 * k, 16)] = jnp.right_shift(v, 14)

    def wait_g(r):
        pltpu.make_async_copy(tbl_s.at[idxb.at[r, 0]], rows.at[r],
                              gsem.at[r]).wait()

    def start_g(r):
        pltpu.async_copy(tbl_s.at[idxb.at[r, 0]], rows.at[r], gsem.at[r])

    def start_s(r):
        pltpu.async_copy(rows.at[r], acc.at[idxb.at[r, 1]], ssem.at[r],
                         add=True)

    def wait_s(r):
        pltpu.make_async_copy(rows.at[r], acc.at[idxb.at[r, 1]],
                              ssem.at[r]).wait()

    H = _R // 2

    def step(j, k):
        wait_g(k)
        start_s(k)
        wait_s((k + H) % _R)
        unpack((k + H) % _R, j + H)
        start_g((k + H) % _R)

    for j in range(H):
        unpack(j, j)
        start_g(j)
    for j in range(H):
        wait_g(j)
        start_s(j)
        unpack(j + H, j + H)
        start_g(j + H)
    for j in range(H, _R):
        step(j, j)

    @pl.loop(_R, _CH - _R, step=_R)
    def _(g):
        for k in range(_R):
            step(g + k, k)

    for j in range(_CH - _R, _CH):
        step(j, j % _R)
    for j in range(_CH - H, _CH):
        wait_s(j % _R)
    for j in range(_CH, _CH + H):
        wait_g(j % _R)
    plsc.subcore_barrier()
    pltpu.sync_copy(acc.at[pl.ds(s * _RPT, _RPT)],
                    out_hbm.at[c, pl.ds(s * _RPT, _RPT)])


@functools.partial(
    pl.kernel,
    out_type=jax.ShapeDtypeStruct((2, _NT, _D), jnp.float32),
    scratch_types=[
        pltpu.VMEM((_CH + 2, _CHW), jnp.int32),
        pltpu.VMEM((_R, 2, _CHW), jnp.int32),
        pltpu.VMEM((_R, _CHW, _D), jnp.float32),
        pltpu.MemorySpace.VMEM_SHARED((_NT, _D), jnp.float32),
        pltpu.MemorySpace.VMEM_SHARED((_N, _D), jnp.float32),
        pltpu.SemaphoreType.DMA((_R,)),
        pltpu.SemaphoreType.DMA((_R,)),
        pltpu.SemaphoreType.DMA,
    ],
    **_MESH,
)
def _edge_pass(*args):
    _edge_pass_body(*args)


def _hist_body(d0_hbm, d1_hbm, d2_hbm, zeros_hbm, ones_hbm, out_hbm,
               idx, ones_v, a0, a1, a2):
    c = lax.axis_index("c")
    s = lax.axis_index("s")
    w = s * 2 + c
    accs = (a0, a1, a2)
    srcs = (d0_hbm, d1_hbm, d2_hbm)
    pltpu.sync_copy(ones_hbm, ones_v)
    for acc in accs:
        pltpu.sync_copy(zeros_hbm, acc.at[pl.ds(s * _HPT, _HPT)])
    plsc.subcore_barrier()
    for acc, dh in zip(accs, srcs):
        pltpu.sync_copy(dh.at[w, pl.ds(0, _CH)], idx)

        @pl.loop(0, _CH)
        def _(r):
            for k in range(_CHW // 16):
                idx[r, pl.ds(16 * k, 16)] = jnp.right_shift(
                    idx[r, pl.ds(16 * k, 16)], 14)

        @pl.loop(0, _CH)
        def _(g, acc=acc):
            pltpu.sync_copy(ones_v, acc.at[idx.at[g]], add=True)

    plsc.subcore_barrier()
    for h, acc in enumerate(accs):
        pltpu.sync_copy(acc.at[pl.ds(s * _HPT, _HPT)],
                        out_hbm.at[h, c, pl.ds(s * _HPT, _HPT)])


@functools.partial(
    pl.kernel,
    out_type=jax.ShapeDtypeStruct((3, 2, _NT), jnp.float32),
    scratch_types=[
        pltpu.VMEM((_CH, _CHW), jnp.int32),
        pltpu.VMEM((_CHW,), jnp.float32),
        pltpu.MemorySpace.VMEM_SHARED((_NT,), jnp.float32),
        pltpu.MemorySpace.VMEM_SHARED((_NT,), jnp.float32),
        pltpu.MemorySpace.VMEM_SHARED((_NT,), jnp.float32),
    ],
    **_MESH,
)
def _hist(*args):
    _hist_body(*args)


_BLK = 2000


def _tc(body, n_out_cols, *arrs, blk=_BLK):
    grid = (_N // blk,)
    in_specs = []
    for a in arrs:
        if a.ndim == 3:
            in_specs.append(pl.BlockSpec((2, blk, a.shape[2]), lambda i: (0, i, 0)))
        elif a.shape[0] == _N:
            in_specs.append(pl.BlockSpec((blk, a.shape[1]), lambda i: (i, 0)))
        else:
            zero_map = (lambda n: (lambda i: (0,) * n))(a.ndim)
            in_specs.append(pl.BlockSpec(a.shape, zero_map))
    return pl.pallas_call(
        body,
        out_shape=jax.ShapeDtypeStruct((_N, n_out_cols), jnp.float32),
        grid=grid,
        in_specs=in_specs,
        out_specs=pl.BlockSpec((blk, n_out_cols), lambda i: (i, 0)),
    )(*arrs)


def _mm_body(x_ref, w_ref, o_ref):
    o_ref[...] = jnp.dot(x_ref[...], w_ref[...],
                         preferred_element_type=jnp.float32)


def _scal_body(h_ref, o_ref):
    h = h_ref[...]
    tot = h[:, 0, :] + h[:, 1, :]
    d = tot[0:1, :]
    b = tot[1:2, :]
    deg = tot[2:3, :] + 1.0
    o_ref[0:1, :] = jnp.where(d > 0, 1.0 / d, 0.0)
    o_ref[1:2, :] = jnp.where(b > 0, 1.0 / b, 0.0)
    o_ref[2:3, :] = lax.rsqrt(deg)


def _combine_scale_body(p_ref, v_ref, o_ref):
    o_ref[...] = v_ref[...] * (p_ref[0] + p_ref[1])


def _hyper_out_mm_body(p_ref, dinv_ref, b_ref, w_ref, o_ref):
    h = jnp.maximum(dinv_ref[...] * (p_ref[0] + p_ref[1]) + b_ref[...], 0.0)
    o_ref[...] = jnp.dot(h, w_ref[...], preferred_element_type=jnp.float32)


def _concat_mm_body(p_ref, dinv_ref, dis_ref, b_ref, x_ref, wt_ref, wb_ref, o_ref):
    h2 = dinv_ref[...] * (p_ref[0] + p_ref[1]) + b_ref[...]
    xwc = (jnp.dot(x_ref[...], wt_ref[...], preferred_element_type=jnp.float32)
           + jnp.dot(h2, wb_ref[...], preferred_element_type=jnp.float32))
    o_ref[...] = dis_ref[...] * xwc


def _gcn1_out_body(p_ref, xws_ref, dis_ref, b_ref, w_ref, o_ref):
    g = jnp.maximum(dis_ref[...] * (p_ref[0] + p_ref[1] + xws_ref[...])
                    + b_ref[...], 0.0)
    o_ref[...] = dis_ref[...] * jnp.dot(g, w_ref[...],
                                        preferred_element_type=jnp.float32)


def _gcn2_out_body(p_ref, xgs_ref, dis_ref, b2_ref, wlp_ref, blp_ref, o_ref):
    g2 = dis_ref[...] * (p_ref[0] + p_ref[1] + xgs_ref[...]) + b2_ref[...]
    o_ref[...] = (jnp.dot(g2, wlp_ref[...], preferred_element_type=jnp.float32)
                  + blp_ref[...])


def _prep_pair(src, dst):
    pad_src = jnp.arange(_CAP, dtype=jnp.int32) % _N
    pad_dst = _N + (jnp.arange(_CAP, dtype=jnp.int32) % (_NT - _N))
    pair = (pad_src + pad_dst * 16384).at[:_E].set(src + dst * 16384)
    pair = pair.reshape(_NW, _CH, _CHW)
    extra = jnp.arange(_NW * 2 * _CHW, dtype=jnp.int32) % _N
    return jnp.concatenate(
        [pair, extra.reshape(_NW, 2, _CHW)], axis=1)


def kernel(x, edge_index, hyperedge_index, W_h1, b_h1, W_h2, b_h2,
           W_c1, b_c1, W_c2, b_c2, W_lp, b_lp):
    f32 = jnp.float32
    pA = _prep_pair(hyperedge_index[0], hyperedge_index[1])
    pB = _prep_pair(hyperedge_index[1], hyperedge_index[0])
    pG = _prep_pair(edge_index[0], edge_index[1])

    zeros2d = jnp.zeros((8, _D), f32)
    zeros1d = jnp.zeros((_HPT,), f32)
    ones1d = jnp.ones((_CHW,), f32)

    b_h1r = b_h1.reshape(1, _D)
    b_h2r = b_h2.reshape(1, _D)
    b_c1r = b_c1.reshape(1, _D)
    b_c2r = jnp.zeros((1, _D), f32).at[0, :_C].set(b_c2)
    b_lpr = b_lp.reshape(1, _C)
    Wc1_top = W_c1[:_F]
    Wc1_bot = W_c1[_F:]
    W_c2p = jnp.zeros((_D, _D), f32).at[:, :_C].set(W_c2)
    W_lpp = jnp.zeros((_D, _C), f32).at[:_C].set(W_lp)

    hist = _hist(pB, pA, pG, zeros1d, ones1d)
    scal = pl.pallas_call(
        _scal_body,
        out_shape=jax.ShapeDtypeStruct((3, _NT), f32),
        in_specs=[pl.BlockSpec((3, 2, _NT), lambda: (0, 0, 0))],
        out_specs=pl.BlockSpec((3, _NT), lambda: (0, 0)),
    )(hist)
    dinv = scal[0, :_N].reshape(_N, 1)
    binv = scal[1, :_N].reshape(_N, 1)
    dis = scal[2, :_N].reshape(_N, 1)

    xw1 = _tc(_mm_body, _D, x, W_h1)
    pA1 = _edge_pass(xw1, pA, zeros2d)
    ef1 = _tc(_combine_scale_body, _D, pA1, binv)
    pB1 = _edge_pass(ef1, pB, zeros2d)

    xw2 = _tc(_hyper_out_mm_body, _D, pB1, dinv, b_h1r, W_h2)
    pA2 = _edge_pass(xw2, pA, zeros2d)
    ef2 = _tc(_combine_scale_body, _D, pA2, binv)
    pB2 = _edge_pass(ef2, pB, zeros2d)

    xws = _tc(_concat_mm_body, _D, pB2, dinv, dis, b_h2r, x, Wc1_top, Wc1_bot)
    pG1 = _edge_pass(xws, pG, zeros2d)

    xgs = _tc(_gcn1_out_body, _D, pG1, xws, dis, b_c1r, W_c2p)
    pG2 = _edge_pass(xgs, pG, zeros2d)

    return _tc(_gcn2_out_body, _C, pG2, xgs, dis, b_c2r, W_lpp, b_lpr)


# speedup vs baseline: 1.8995x; 1.8995x over previous
import functools

import jax
import jax.numpy as jnp
from jax import lax
from jax.experimental import pallas as pl
from jax.experimental.pallas import tpu as pltpu
from jax.experimental.pallas import tpu_sc as plsc

_N = 10000
_E = 320000
_F = 128
_D = 64
_C = 40
_NW = 32
_CHW = 128
_CH = 80
_CAP = _NW * _CH * _CHW
_NT = 10112
_SINK = 10100
_RPT = _NT // 16
_R = 4
_HPT = _NT // 16

_MESH = dict(
    mesh=plsc.VectorSubcoreMesh(core_axis_name="c", subcore_axis_name="s"),
    compiler_params=pltpu.CompilerParams(use_tc_tiling_on_sc=False),
)


def _edge_pass_body(tbl_hbm, pair_hbm, zeros_hbm, out_hbm,
                    idx_p, idxb, rows, acc, tbl_s, gsem, ssem, zsem):
    c = lax.axis_index("c")
    s = lax.axis_index("s")
    w = s * 2 + c
    pltpu.sync_copy(pair_hbm.at[w], idx_p)

    pltpu.async_copy(zeros_hbm, acc.at[pl.ds(s * _RPT, _RPT)], zsem)

    pltpu.sync_copy(tbl_hbm.at[pl.ds(s * 624, 624)],
                    tbl_s.at[pl.ds(s * 624, 624)])

    @pl.when(s == 15)
    def _():
        pltpu.sync_copy(tbl_hbm.at[pl.ds(9984, 16)],
                        tbl_s.at[pl.ds(9984, 16)])

    pltpu.make_async_copy(zeros_hbm, acc.at[pl.ds(s * _RPT, _RPT)],
                          zsem).wait()
    plsc.subcore_barrier()

    def unpack(r, j):
        for k in range(_CHW // 16):
            v = idx_p[j, pl.ds(16 * k, 16)]
            idxb[r, 0, pl.ds(16 * k, 16)] = jnp.bitwise_and(v, 16383)
            idxb[r, 1, pl.ds(16 * k, 16)] = jnp.right_shift(v, 14)

    def wait_g(r):
        pltpu.make_async_copy(tbl_s.at[idxb.at[r, 0]], rows.at[r],
                              gsem.at[r]).wait()

    def start_g(r):
        pltpu.async_copy(tbl_s.at[idxb.at[r, 0]], rows.at[r], gsem.at[r])

    def start_s(r):
        pltpu.async_copy(rows.at[r], acc.at[idxb.at[r, 1]], ssem.at[r],
                         add=True)

    def wait_s(r):
        pltpu.make_async_copy(rows.at[r], acc.at[idxb.at[r, 1]],
                              ssem.at[r]).wait()

    H = _R // 2

    def step(j, k):
        wait_g(k)
        start_s(k)
        wait_s((k + H) % _R)
        unpack((k + H) % _R, j + H)
        start_g((k + H) % _R)

    for j in range(H):
        unpack(j, j)
        start_g(j)
    for j in range(H):
        wait_g(j)
        start_s(j)
        unpack(j + H, j + H)
        start_g(j + H)
    for j in range(H, _R):
        step(j, j)

    @pl.loop(_R, _CH - _R, step=_R)
    def _(g):
        for k in range(_R):
            step(g + k, k)

    for j in range(_CH - _R, _CH):
        step(j, j % _R)
    for j in range(_CH - H, _CH):
        wait_s(j % _R)
    for j in range(_CH, _CH + H):
        wait_g(j % _R)
    plsc.subcore_barrier()
    pltpu.sync_copy(acc.at[pl.ds(s * _RPT, _RPT)],
                    out_hbm.at[c, pl.ds(s * _RPT, _RPT)])


@functools.partial(
    pl.kernel,
    out_type=jax.ShapeDtypeStruct((2, _NT, _D), jnp.float32),
    scratch_types=[
        pltpu.VMEM((_CH + 2, _CHW), jnp.int32),
        pltpu.VMEM((_R, 2, _CHW), jnp.int32),
        pltpu.VMEM((_R, _CHW, _D), jnp.float32),
        pltpu.MemorySpace.VMEM_SHARED((_NT, _D), jnp.float32),
        pltpu.MemorySpace.VMEM_SHARED((_N, _D), jnp.float32),
        pltpu.SemaphoreType.DMA((_R,)),
        pltpu.SemaphoreType.DMA((_R,)),
        pltpu.SemaphoreType.DMA,
    ],
    **_MESH,
)
def _edge_pass(*args):
    _edge_pass_body(*args)


def _hist_body(d0_hbm, d1_hbm, d2_hbm, zeros_hbm, ones_hbm, out_hbm,
               idx, ones_v, a0, a1, a2, hsem):
    c = lax.axis_index("c")
    s = lax.axis_index("s")
    w = s * 2 + c
    accs = (a0, a1, a2)
    srcs = (d0_hbm, d1_hbm, d2_hbm)
    pltpu.sync_copy(ones_hbm, ones_v)
    for acc in accs:
        pltpu.sync_copy(zeros_hbm, acc.at[pl.ds(s * _HPT, _HPT)])
    plsc.subcore_barrier()
    for acc, dh in zip(accs, srcs):
        pltpu.sync_copy(dh.at[w, pl.ds(0, _CH)], idx)

        @pl.loop(0, _CH)
        def _(r):
            for k in range(_CHW // 16):
                idx[r, pl.ds(16 * k, 16)] = jnp.right_shift(
                    idx[r, pl.ds(16 * k, 16)], 14)

        @pl.loop(0, _CH, step=2)
        def _(g, acc=acc):
            pltpu.async_copy(ones_v, acc.at[idx.at[g]], hsem.at[0], add=True)
            pltpu.async_copy(ones_v, acc.at[idx.at[g + 1]], hsem.at[1],
                             add=True)
            pltpu.make_async_copy(ones_v, acc.at[idx.at[g]], hsem.at[0]).wait()
            pltpu.make_async_copy(ones_v, acc.at[idx.at[g + 1]],
                                  hsem.at[1]).wait()

    plsc.subcore_barrier()
    for h, acc in enumerate(accs):
        pltpu.sync_copy(acc.at[pl.ds(s * _HPT, _HPT)],
                        out_hbm.at[h, c, pl.ds(s * _HPT, _HPT)])


@functools.partial(
    pl.kernel,
    out_type=jax.ShapeDtypeStruct((3, 2, _NT), jnp.float32),
    scratch_types=[
        pltpu.VMEM((_CH, _CHW), jnp.int32),
        pltpu.VMEM((_CHW,), jnp.float32),
        pltpu.MemorySpace.VMEM_SHARED((_NT,), jnp.float32),
        pltpu.MemorySpace.VMEM_SHARED((_NT,), jnp.float32),
        pltpu.MemorySpace.VMEM_SHARED((_NT,), jnp.float32),
        pltpu.SemaphoreType.DMA((2,)),
    ],
    **_MESH,
)
def _hist(*args):
    _hist_body(*args)


_BLK = 2000


def _tc(body, n_out_cols, *arrs, blk=_BLK):
    grid = (_N // blk,)
    in_specs = []
    for a in arrs:
        if a.ndim == 3:
            in_specs.append(pl.BlockSpec((2, blk, a.shape[2]), lambda i: (0, i, 0)))
        elif a.shape[0] == _N:
            in_specs.append(pl.BlockSpec((blk, a.shape[1]), lambda i: (i, 0)))
        else:
            zero_map = (lambda n: (lambda i: (0,) * n))(a.ndim)
            in_specs.append(pl.BlockSpec(a.shape, zero_map))
    return pl.pallas_call(
        body,
        out_shape=jax.ShapeDtypeStruct((_N, n_out_cols), jnp.float32),
        grid=grid,
        in_specs=in_specs,
        out_specs=pl.BlockSpec((blk, n_out_cols), lambda i: (i, 0)),
    )(*arrs)


def _mm_body(x_ref, w_ref, o_ref):
    o_ref[...] = jnp.dot(x_ref[...], w_ref[...],
                         preferred_element_type=jnp.float32)


def _scal_body(h_ref, o_ref):
    h = h_ref[...]
    tot = h[:, 0, :] + h[:, 1, :]
    d = tot[0:1, :]
    b = tot[1:2, :]
    deg = tot[2:3, :] + 1.0
    o_ref[0:1, :] = jnp.where(d > 0, 1.0 / d, 0.0)
    o_ref[1:2, :] = jnp.where(b > 0, 1.0 / b, 0.0)
    o_ref[2:3, :] = lax.rsqrt(deg)


def _combine_scale_body(p_ref, v_ref, o_ref):
    o_ref[...] = v_ref[...] * (p_ref[0] + p_ref[1])


def _hyper_out_mm_body(p_ref, dinv_ref, b_ref, w_ref, o_ref):
    h = jnp.maximum(dinv_ref[...] * (p_ref[0] + p_ref[1]) + b_ref[...], 0.0)
    o_ref[...] = jnp.dot(h, w_ref[...], preferred_element_type=jnp.float32)


def _concat_mm_body(p_ref, dinv_ref, dis_ref, b_ref, x_ref, wt_ref, wb_ref, o_ref):
    h2 = dinv_ref[...] * (p_ref[0] + p_ref[1]) + b_ref[...]
    xwc = (jnp.dot(x_ref[...], wt_ref[...], preferred_element_type=jnp.float32)
           + jnp.dot(h2, wb_ref[...], preferred_element_type=jnp.float32))
    o_ref[...] = dis_ref[...] * xwc


def _gcn1_out_body(p_ref, xws_ref, dis_ref, b_ref, w_ref, o_ref):
    g = jnp.maximum(dis_ref[...] * (p_ref[0] + p_ref[1] + xws_ref[...])
                    + b_ref[...], 0.0)
    o_ref[...] = dis_ref[...] * jnp.dot(g, w_ref[...],
                                        preferred_element_type=jnp.float32)


def _gcn2_out_body(p_ref, xgs_ref, dis_ref, b2_ref, wlp_ref, blp_ref, o_ref):
    g2 = dis_ref[...] * (p_ref[0] + p_ref[1] + xgs_ref[...]) + b2_ref[...]
    o_ref[...] = (jnp.dot(g2, wlp_ref[...], preferred_element_type=jnp.float32)
                  + blp_ref[...])


def _prep_pair(src, dst):
    pad_src = jnp.arange(_CAP, dtype=jnp.int32) % _N
    pad_dst = _N + (jnp.arange(_CAP, dtype=jnp.int32) % (_NT - _N))
    pair = (pad_src + pad_dst * 16384).at[:_E].set(src + dst * 16384)
    pair = pair.reshape(_NW, _CH, _CHW)
    extra = jnp.arange(_NW * 2 * _CHW, dtype=jnp.int32) % _N
    return jnp.concatenate(
        [pair, extra.reshape(_NW, 2, _CHW)], axis=1)


def kernel(x, edge_index, hyperedge_index, W_h1, b_h1, W_h2, b_h2,
           W_c1, b_c1, W_c2, b_c2, W_lp, b_lp):
    f32 = jnp.float32
    pA = _prep_pair(hyperedge_index[0], hyperedge_index[1])
    pB = _prep_pair(hyperedge_index[1], hyperedge_index[0])
    pG = _prep_pair(edge_index[0], edge_index[1])

    zeros2d = jnp.zeros((_RPT, _D), f32)
    zeros1d = jnp.zeros((_HPT,), f32)
    ones1d = jnp.ones((_CHW,), f32)

    b_h1r = b_h1.reshape(1, _D)
    b_h2r = b_h2.reshape(1, _D)
    b_c1r = b_c1.reshape(1, _D)
    b_c2r = jnp.zeros((1, _D), f32).at[0, :_C].set(b_c2)
    b_lpr = b_lp.reshape(1, _C)
    Wc1_top = W_c1[:_F]
    Wc1_bot = W_c1[_F:]
    W_c2p = jnp.zeros((_D, _D), f32).at[:, :_C].set(W_c2)
    W_lpp = jnp.zeros((_D, _C), f32).at[:_C].set(W_lp)

    hist = _hist(pB, pA, pG, zeros1d, ones1d)
    scal = pl.pallas_call(
        _scal_body,
        out_shape=jax.ShapeDtypeStruct((3, _NT), f32),
        in_specs=[pl.BlockSpec((3, 2, _NT), lambda: (0, 0, 0))],
        out_specs=pl.BlockSpec((3, _NT), lambda: (0, 0)),
    )(hist)
    dinv = scal[0, :_N].reshape(_N, 1)
    binv = scal[1, :_N].reshape(_N, 1)
    dis = scal[2, :_N].reshape(_N, 1)

    xw1 = _tc(_mm_body, _D, x, W_h1)
    pA1 = _edge_pass(xw1, pA, zeros2d)
    ef1 = _tc(_combine_scale_body, _D, pA1, binv)
    pB1 = _edge_pass(ef1, pB, zeros2d)

    xw2 = _tc(_hyper_out_mm_body, _D, pB1, dinv, b_h1r, W_h2)
    pA2 = _edge_pass(xw2, pA, zeros2d)
    ef2 = _tc(_combine_scale_body, _D, pA2, binv)
    pB2 = _edge_pass(ef2, pB, zeros2d)

    xws = _tc(_concat_mm_body, _D, pB2, dinv, dis, b_h2r, x, Wc1_top, Wc1_bot)
    pG1 = _edge_pass(xws, pG, zeros2d)

    xgs = _tc(_gcn1_out_body, _D, pG1, xws, dis, b_c1r, W_c2p)
    pG2 = _edge_pass(xgs, pG, zeros2d)

    return _tc(_gcn2_out_body, _C, pG2, xgs, dis, b_c2r, W_lpp, b_lpr)
